# Initial kernel scaffold; baseline (speedup 1.0000x reference)
import jax
import jax.numpy as jnp
from jax.experimental import pallas as pl


def kernel(x, edge_index, edge_weight, W1, b1, W2, b2, county_bias, W_ih, W_hh, b_ih, b_hh, Wm1, bm1, Wm2, bm2):
    raise NotImplementedError("write your pallas kernel here")


# speedup vs baseline: 3.2921x; 3.2921x over previous
import functools

import jax
import jax.numpy as jnp
from jax import lax
from jax.experimental import pallas as pl
from jax.experimental.pallas import tpu as pltpu
from jax.experimental.pallas import tpu_sc as plsc

N = 10000
NP = 10240
E = 320000
T = 24
FEAT = 15
FPAD = 16
EMB = 64
H = 128
FC = 128
NCH1 = 3
NCH2 = 12
EB = 80
BN = 512
RPT = NP // 16


def _sc_deg(cols, w, zeros1d):
    ept = E // 32
    nb = ept // EB
    mesh = plsc.VectorSubcoreMesh(core_axis_name="c", subcore_axis_name="s")

    @functools.partial(
        pl.kernel, mesh=mesh,
        out_type=jax.ShapeDtypeStruct((32, NP), jnp.float32),
        compiler_params=pltpu.CompilerParams(needs_layout_passes=False),
        scratch_types=[
            pltpu.VMEM((EB,), jnp.int32),
            pltpu.VMEM((EB,), jnp.float32),
            pltpu.VMEM((NP,), jnp.float32),
        ],
    )
    def k(cols_hbm, w_hbm, z_hbm, out_hbm, colb, wb, deg_l):
        cc = lax.axis_index("c")
        ss = lax.axis_index("s")
        wid = ss * 2 + cc
        base = wid * ept
        pltpu.sync_copy(z_hbm, deg_l)

        def body(b, carry):
            start = base + b * EB
            pltpu.sync_copy(cols_hbm.at[pl.ds(start, EB)], colb)
            pltpu.sync_copy(w_hbm.at[pl.ds(start, EB)], wb)
            for g in range(EB // 16):
                ci = colb[pl.ds(g * 16, 16)]
                wv = wb[pl.ds(g * 16, 16)]
                plsc.addupdate_scatter(deg_l, [ci], wv)
            return carry

        lax.fori_loop(0, nb, body, 0)
        pltpu.sync_copy(deg_l, out_hbm.at[wid])

    return k(cols, w, zeros1d)


def _sc_spmm(table, rows, cols, w, zeros2d, nch):
    n0 = (nch + 1) // 2
    ept = E // 16
    nb = ept // EB
    mesh = plsc.VectorSubcoreMesh(core_axis_name="c", subcore_axis_name="s")

    @functools.partial(
        pl.kernel, mesh=mesh,
        out_type=jax.ShapeDtypeStruct((nch, NP, FC), jnp.float32),
        compiler_params=pltpu.CompilerParams(needs_layout_passes=False),
        scratch_types=[
            pltpu.VMEM((EB,), jnp.int32),
            pltpu.VMEM((EB,), jnp.int32),
            pltpu.VMEM((EB,), jnp.float32),
            pltpu.VMEM((EB,), jnp.int32),
            pltpu.VMEM((EB, FC), jnp.float32),
            pltpu.VMEM_SHARED((NP, FC), jnp.float32),
            pltpu.SemaphoreType.DMA,
        ],
    )
    def k(tab_hbm, rows_hbm, cols_hbm, w_hbm, z_hbm, out_hbm,
          rowb, colb, wb, idxb, buf, acc, sem):
        cc = lax.axis_index("c")
        ss = lax.axis_index("s")
        ebase = ss * ept
        iota16 = lax.iota(jnp.int32, 16)
        one16 = jnp.ones((16,), jnp.int32)

        def chunk_body(j, carry0):
            chunk_g = cc * n0 + j
            pltpu.sync_copy(z_hbm, acc.at[pl.ds(ss * RPT, RPT)])
            plsc.subcore_barrier()

            def body(b, carry):
                start = ebase + b * EB
                pltpu.sync_copy(rows_hbm.at[pl.ds(start, EB)], rowb)
                pltpu.sync_copy(cols_hbm.at[pl.ds(start, EB)], colb)
                pltpu.sync_copy(w_hbm.at[pl.ds(start, EB)], wb)
                for g in range(EB // 16):
                    rv = rowb[pl.ds(g * 16, 16)]
                    idxb[pl.ds(g * 16, 16)] = rv * nch + chunk_g
                pltpu.async_copy(tab_hbm.at[idxb], buf, sem).wait()
                for g in range(EB // 16):
                    wv = wb[pl.ds(g * 16, 16)]
                    rowids = g * 16 + iota16
                    colv = jnp.zeros((16,), jnp.int32)
                    for _f in range(FC):
                        v = plsc.load_gather(buf, [rowids, colv])
                        plsc.store_scatter(buf, [rowids, colv], v * wv)
                        colv = colv + one16
                pltpu.sync_copy(buf, acc.at[colb], add=True)
                return carry

            lax.fori_loop(0, nb, body, 0)
            plsc.subcore_barrier()
            pltpu.sync_copy(acc.at[pl.ds(ss * RPT, RPT)],
                            out_hbm.at[chunk_g, pl.ds(ss * RPT, RPT)])
            return carry0

        nloc = jnp.where(cc == 0, n0, nch - n0)
        lax.fori_loop(0, nloc, chunk_body, 0)

    return k(table, rows, cols, w, zeros2d)


def _tc_prep(pdeg, Xt):
    def kfn(pd_ref, xt_ref, dinv_ref, t1_ref):
        d = jnp.sum(pd_ref[...], axis=0) + 1.0
        dv = lax.rsqrt(d)
        dinv_ref[...] = dv
        t1_ref[...] = xt_ref[...] * dv[:, None, None]

    return pl.pallas_call(
        kfn,
        grid=(NP // BN,),
        in_specs=[pl.BlockSpec((32, BN), lambda i: (0, i)),
                  pl.BlockSpec((BN, NCH1, FC), lambda i: (i, 0, 0))],
        out_specs=[pl.BlockSpec((BN,), lambda i: (i,)),
                   pl.BlockSpec((BN, NCH1, FC), lambda i: (i, 0, 0))],
        out_shape=[jax.ShapeDtypeStruct((NP,), jnp.float32),
                   jax.ShapeDtypeStruct((NP, NCH1, FC), jnp.float32)],
    )(pdeg, Xt)


def _tc_mid(Y, t1s, dinv, W1p, b1r, W2):
    def kfn(y_ref, x_ref, dv_ref, w1_ref, b1_ref, w2_ref, o_ref):
        dv = dv_ref[...][:, None]
        w1 = w1_ref[...]
        b1v = b1_ref[...]
        w2 = w2_ref[...]
        tpc1 = FC // FPAD
        tpc2 = FC // EMB
        for t in range(T):
            c1, o1 = t // tpc1, (t % tpc1) * FPAD
            ax = (y_ref[c1, :, o1:o1 + FPAD] + x_ref[:, c1, o1:o1 + FPAD]) * dv
            h1 = jnp.maximum(
                jnp.dot(ax, w1, preferred_element_type=jnp.float32) + b1v, 0.0)
            p = jnp.dot(h1, w2, preferred_element_type=jnp.float32)
            o2 = (t % tpc2) * EMB
            o_ref[:, t // tpc2, o2:o2 + EMB] = p * dv

    return pl.pallas_call(
        kfn,
        grid=(NP // BN,),
        in_specs=[pl.BlockSpec((NCH1, BN, FC), lambda i: (0, i, 0)),
                  pl.BlockSpec((BN, NCH1, FC), lambda i: (i, 0, 0)),
                  pl.BlockSpec((BN,), lambda i: (i,)),
                  pl.BlockSpec((FPAD, EMB), lambda i: (0, 0)),
                  pl.BlockSpec((1, EMB), lambda i: (0, 0)),
                  pl.BlockSpec((EMB, EMB), lambda i: (0, 0))],
        out_specs=pl.BlockSpec((BN, NCH2, FC), lambda i: (i, 0, 0)),
        out_shape=jax.ShapeDtypeStruct((NP, NCH2, FC), jnp.float32),
    )(Y, t1s, dinv, W1p, b1r, W2)


def _tc_final(Z, t2s, dinv, cbp, b2r, WihT, WhhT, bihr, bhhr,
              Wm1T, bm1r, Wm2T, bm2r):
    def kfn(z_ref, p_ref, dv_ref, cb_ref, b2_ref, wih_ref, whh_ref,
            bi_ref, bh_ref, wm1_ref, bm1_ref, wm2_ref, bm2_ref, o_ref):
        dv = dv_ref[...][:, None]
        add_t = b2_ref[...] + cb_ref[...]
        bias = bi_ref[...] + bh_ref[...]
        wih = wih_ref[...]
        whh = whh_ref[...]
        h = jnp.zeros((BN, H), jnp.float32)
        c = jnp.zeros((BN, H), jnp.float32)
        tpc2 = FC // EMB
        for t in range(T):
            sl = (t % tpc2) * EMB
            e = (z_ref[t // tpc2, :, sl:sl + EMB]
                 + p_ref[:, t // tpc2, sl:sl + EMB]) * dv + add_t
            g = (jnp.dot(e, wih, preferred_element_type=jnp.float32)
                 + jnp.dot(h, whh, preferred_element_type=jnp.float32) + bias)
            i_g = jax.nn.sigmoid(g[:, 0:H])
            f_g = jax.nn.sigmoid(g[:, H:2 * H])
            g_g = jnp.tanh(g[:, 2 * H:3 * H])
            o_g = jax.nn.sigmoid(g[:, 3 * H:4 * H])
            c = f_g * c + i_g * g_g
            h = o_g * jnp.tanh(c)
        z2 = jnp.maximum(
            jnp.dot(h, wm1_ref[...], preferred_element_type=jnp.float32)
            + bm1_ref[...], 0.0)
        pred = (jnp.dot(z2, wm2_ref[...], preferred_element_type=jnp.float32)
                + bm2_ref[...])
        o_ref[...] = pred[:, 0]

    return pl.pallas_call(
        kfn,
        grid=(NP // BN,),
        in_specs=[pl.BlockSpec((NCH2, BN, FC), lambda i: (0, i, 0)),
                  pl.BlockSpec((BN, NCH2, FC), lambda i: (i, 0, 0)),
                  pl.BlockSpec((BN,), lambda i: (i,)),
                  pl.BlockSpec((BN, EMB), lambda i: (i, 0)),
                  pl.BlockSpec((1, EMB), lambda i: (0, 0)),
                  pl.BlockSpec((EMB, 4 * H), lambda i: (0, 0)),
                  pl.BlockSpec((H, 4 * H), lambda i: (0, 0)),
                  pl.BlockSpec((1, 4 * H), lambda i: (0, 0)),
                  pl.BlockSpec((1, 4 * H), lambda i: (0, 0)),
                  pl.BlockSpec((H, H // 2), lambda i: (0, 0)),
                  pl.BlockSpec((1, H // 2), lambda i: (0, 0)),
                  pl.BlockSpec((H // 2, 1), lambda i: (0, 0)),
                  pl.BlockSpec((1, 1), lambda i: (0, 0))],
        out_specs=pl.BlockSpec((BN,), lambda i: (i,)),
        out_shape=jax.ShapeDtypeStruct((NP,), jnp.float32),
    )(Z, t2s, dinv, cbp, b2r, WihT, WhhT, bihr, bhhr, Wm1T, bm1r, Wm2T, bm2r)


def kernel(x, edge_index, edge_weight, W1, b1, W2, b2, county_bias,
           W_ih, W_hh, b_ih, b_hh, Wm1, bm1, Wm2, bm2):
    rows = edge_index[0].astype(jnp.int32)
    cols = edge_index[1].astype(jnp.int32)
    w = edge_weight.astype(jnp.float32)

    xp = jnp.pad(x, ((0, 0), (0, NP - N), (0, FPAD - FEAT)))
    Xt = xp.transpose(1, 0, 2).reshape(NP, NCH1, FC)
    z1 = jnp.zeros((NP,), jnp.float32)
    z2 = jnp.zeros((RPT, FC), jnp.float32)

    pdeg = _sc_deg(cols, w, z1)
    dinv, t1s = _tc_prep(pdeg, Xt)

    tab1 = t1s.reshape(NP * NCH1, FC)
    Y = _sc_spmm(tab1, rows, cols, w, z2, NCH1)

    W1p = jnp.pad(W1, ((0, FPAD - FEAT), (0, 0)))
    t2s = _tc_mid(Y, t1s, dinv, W1p, b1[None], W2)

    tab2 = t2s.reshape(NP * NCH2, FC)
    Z = _sc_spmm(tab2, rows, cols, w, z2, NCH2)

    cbp = jnp.pad(county_bias, ((0, NP - N), (0, 0)))
    preds = _tc_final(Z, t2s, dinv, cbp, b2[None], W_ih.T, W_hh.T,
                      b_ih[None], b_hh[None], Wm1.T, bm1[None], Wm2.T,
                      bm2[None])
    return preds[:N]


# speedup vs baseline: 16.7793x; 5.0968x over previous
import functools

import jax
import jax.numpy as jnp
from jax import lax
from jax.experimental import pallas as pl
from jax.experimental.pallas import tpu as pltpu
from jax.experimental.pallas import tpu_sc as plsc

N = 10000
NP = 10240
E = 320000
T = 24
FEAT = 15
FPAD = 16
EMB = 64
H = 128
FC = 128
NCH1 = 3
NCH2 = 12
EB = 80
BB = 50
NACC = 10112
BN = 512
RPT = NACC // 16
PK = 16384


def _sc_deg(rows, cols, w, zeros1d):
    ept = E // 32
    nb = ept // EB
    mesh = plsc.VectorSubcoreMesh(core_axis_name="c", subcore_axis_name="s")

    @functools.partial(
        pl.kernel, mesh=mesh,
        out_type=[jax.ShapeDtypeStruct((32, NP), jnp.float32),
                  jax.ShapeDtypeStruct((E,), jnp.int32)],
        compiler_params=pltpu.CompilerParams(needs_layout_passes=False),
        scratch_types=[
            pltpu.VMEM((EB,), jnp.int32),
            pltpu.VMEM((EB,), jnp.int32),
            pltpu.VMEM((EB,), jnp.float32),
            pltpu.VMEM((EB,), jnp.int32),
            pltpu.VMEM((NP,), jnp.float32),
        ],
    )
    def k(rows_hbm, cols_hbm, w_hbm, z_hbm, out_hbm, pk_hbm,
          rowb, colb, wb, packb, deg_l):
        cc = lax.axis_index("c")
        ss = lax.axis_index("s")
        wid = ss * 2 + cc
        base = wid * ept
        pltpu.sync_copy(z_hbm, deg_l)

        def body(b, carry):
            start = base + b * EB
            pltpu.sync_copy(rows_hbm.at[pl.ds(start, EB)], rowb)
            pltpu.sync_copy(cols_hbm.at[pl.ds(start, EB)], colb)
            pltpu.sync_copy(w_hbm.at[pl.ds(start, EB)], wb)
            for g in range(EB // 16):
                ci = colb[pl.ds(g * 16, 16)]
                wv = wb[pl.ds(g * 16, 16)]
                plsc.addupdate_scatter(deg_l, [ci], wv)
                rv = rowb[pl.ds(g * 16, 16)]
                packb[pl.ds(g * 16, 16)] = rv * PK + ci
            pltpu.sync_copy(packb, pk_hbm.at[pl.ds(start, EB)])
            return carry

        lax.fori_loop(0, nb, body, 0)
        pltpu.sync_copy(deg_l, out_hbm.at[wid])

    return k(rows, cols, w, zeros1d)


def _sc_spmm(table, packed, w, zeros2d, nch):
    n0 = (nch + 1) // 2
    ept = E // 16
    nbat = ept // EB
    nblk = nbat // BB
    bedg = BB * EB
    mesh = plsc.VectorSubcoreMesh(core_axis_name="c", subcore_axis_name="s")

    @functools.partial(
        pl.kernel, mesh=mesh,
        out_type=jax.ShapeDtypeStruct((nch, NP, FC), jnp.float32),
        compiler_params=pltpu.CompilerParams(needs_layout_passes=False),
        scratch_types=[
            pltpu.VMEM((bedg,), jnp.int32),
            pltpu.VMEM((bedg,), jnp.float32),
            pltpu.VMEM((EB,), jnp.int32),
            pltpu.VMEM((EB,), jnp.int32),
            pltpu.VMEM((EB,), jnp.int32),
            pltpu.VMEM((EB,), jnp.int32),
            pltpu.VMEM((EB, FC), jnp.float32),
            pltpu.VMEM((EB, FC), jnp.float32),
            pltpu.VMEM((EB, FC), jnp.float32),
            pltpu.VMEM((EB, FC), jnp.float32),
            pltpu.VMEM_SHARED((NACC, FC), jnp.float32),
            pltpu.SemaphoreType.DMA,
            pltpu.SemaphoreType.DMA,
            pltpu.SemaphoreType.DMA,
            pltpu.SemaphoreType.DMA,
        ],
    )
    def k(tab_hbm, pk_hbm, w_hbm, z_hbm, out_hbm,
          pk_blk, w_blk, idxb0, idxb1, colb0, colb1,
          buf0, buf1, sbuf0, sbuf1, acc, sem0, sem1, semS0, semS1):
        cc = lax.axis_index("c")
        ss = lax.axis_index("s")

        def make_idx(lb, idxb, colb, chunk_g):
            for g in range(EB // 16):
                pv = pk_blk[pl.ds(lb * EB + g * 16, 16)]
                rv = lax.shift_right_logical(pv, 14)
                cv = lax.bitwise_and(pv, PK - 1)
                idxb[pl.ds(g * 16, 16)] = rv * nch + chunk_g
                colb[pl.ds(g * 16, 16)] = cv

        def scale(lb, buf, sbuf):
            for e in range(EB):
                wsp = plsc.load_gather(
                    w_blk, [jnp.full((16,), lb * EB + e, jnp.int32)])
                for kk in range(FC // 16):
                    v = buf[e, pl.ds(kk * 16, 16)]
                    sbuf[e, pl.ds(kk * 16, 16)] = v * wsp

        def chunk_body(j, carry0):
            chunk_g = cc * n0 + j
            pltpu.sync_copy(z_hbm, acc.at[pl.ds(ss * RPT, RPT)])
            plsc.subcore_barrier()

            def block_body(blk, carry1):
                ebase = ss * ept + blk * bedg
                pltpu.sync_copy(pk_hbm.at[pl.ds(ebase, bedg)], pk_blk)
                pltpu.sync_copy(w_hbm.at[pl.ds(ebase, bedg)], w_blk)
                make_idx(0, idxb0, colb0, chunk_g)
                pltpu.async_copy(tab_hbm.at[idxb0], buf0, sem0)
                make_idx(1, idxb1, colb1, chunk_g)
                pltpu.async_copy(tab_hbm.at[idxb1], buf1, sem1)

                def body(i, carry):
                    a = 2 * i
                    pltpu.make_async_copy(tab_hbm.at[idxb0], buf0, sem0).wait()
                    scale(a, buf0, sbuf0)
                    pltpu.async_copy(sbuf0, acc.at[colb0], semS0, add=True)
                    pltpu.make_async_copy(tab_hbm.at[idxb1], buf1, sem1).wait()
                    scale(a + 1, buf1, sbuf1)
                    pltpu.async_copy(sbuf1, acc.at[colb1], semS1, add=True)
                    pltpu.make_async_copy(sbuf0, acc.at[colb0], semS0).wait()
                    make_idx(lax.rem(a + 2, BB), idxb0, colb0, chunk_g)
                    pltpu.async_copy(tab_hbm.at[idxb0], buf0, sem0)
                    pltpu.make_async_copy(sbuf1, acc.at[colb1], semS1).wait()
                    make_idx(lax.rem(a + 3, BB), idxb1, colb1, chunk_g)
                    pltpu.async_copy(tab_hbm.at[idxb1], buf1, sem1)
                    return carry

                lax.fori_loop(0, BB // 2, body, 0)
                pltpu.make_async_copy(tab_hbm.at[idxb0], buf0, sem0).wait()
                pltpu.make_async_copy(tab_hbm.at[idxb1], buf1, sem1).wait()
                return carry1

            lax.fori_loop(0, nblk, block_body, 0)
            plsc.subcore_barrier()
            pltpu.sync_copy(acc.at[pl.ds(ss * RPT, RPT)],
                            out_hbm.at[chunk_g, pl.ds(ss * RPT, RPT)])
            return carry0

        nloc = jnp.where(cc == 0, n0, nch - n0)
        lax.fori_loop(0, nloc, chunk_body, 0)

    return k(table, packed, w, zeros2d)


def _tc_prep(pdeg, Xt):
    def kfn(pd_ref, xt_ref, dinv_ref, t1_ref):
        d = jnp.sum(pd_ref[...], axis=0) + 1.0
        dv = lax.rsqrt(d)
        dinv_ref[...] = dv
        t1_ref[...] = xt_ref[...] * dv[:, None, None]

    return pl.pallas_call(
        kfn,
        grid=(NP // BN,),
        in_specs=[pl.BlockSpec((32, BN), lambda i: (0, i)),
                  pl.BlockSpec((BN, NCH1, FC), lambda i: (i, 0, 0))],
        out_specs=[pl.BlockSpec((BN,), lambda i: (i,)),
                   pl.BlockSpec((BN, NCH1, FC), lambda i: (i, 0, 0))],
        out_shape=[jax.ShapeDtypeStruct((NP,), jnp.float32),
                   jax.ShapeDtypeStruct((NP, NCH1, FC), jnp.float32)],
    )(pdeg, Xt)


def _tc_mid(Y, t1s, dinv, W1p, b1r, W2):
    def kfn(y_ref, x_ref, dv_ref, w1_ref, b1_ref, w2_ref, o_ref):
        dv = dv_ref[...][:, None]
        w1 = w1_ref[...]
        b1v = b1_ref[...]
        w2 = w2_ref[...]
        tpc1 = FC // FPAD
        tpc2 = FC // EMB
        for t in range(T):
            c1, o1 = t // tpc1, (t % tpc1) * FPAD
            ax = (y_ref[c1, :, o1:o1 + FPAD] + x_ref[:, c1, o1:o1 + FPAD]) * dv
            h1 = jnp.maximum(
                jnp.dot(ax, w1, preferred_element_type=jnp.float32) + b1v, 0.0)
            p = jnp.dot(h1, w2, preferred_element_type=jnp.float32)
            o2 = (t % tpc2) * EMB
            o_ref[:, t // tpc2, o2:o2 + EMB] = p * dv

    return pl.pallas_call(
        kfn,
        grid=(NP // BN,),
        in_specs=[pl.BlockSpec((NCH1, BN, FC), lambda i: (0, i, 0)),
                  pl.BlockSpec((BN, NCH1, FC), lambda i: (i, 0, 0)),
                  pl.BlockSpec((BN,), lambda i: (i,)),
                  pl.BlockSpec((FPAD, EMB), lambda i: (0, 0)),
                  pl.BlockSpec((1, EMB), lambda i: (0, 0)),
                  pl.BlockSpec((EMB, EMB), lambda i: (0, 0))],
        out_specs=pl.BlockSpec((BN, NCH2, FC), lambda i: (i, 0, 0)),
        out_shape=jax.ShapeDtypeStruct((NP, NCH2, FC), jnp.float32),
    )(Y, t1s, dinv, W1p, b1r, W2)


def _tc_final(Z, t2s, dinv, cbp, b2r, WihT, WhhT, bihr, bhhr,
              Wm1T, bm1r, Wm2T, bm2r):
    def kfn(z_ref, p_ref, dv_ref, cb_ref, b2_ref, wih_ref, whh_ref,
            bi_ref, bh_ref, wm1_ref, bm1_ref, wm2_ref, bm2_ref, o_ref):
        dv = dv_ref[...][:, None]
        add_t = b2_ref[...] + cb_ref[...]
        bias = bi_ref[...] + bh_ref[...]
        wih = wih_ref[...]
        whh = whh_ref[...]
        h = jnp.zeros((BN, H), jnp.float32)
        c = jnp.zeros((BN, H), jnp.float32)
        tpc2 = FC // EMB
        for t in range(T):
            sl = (t % tpc2) * EMB
            e = (z_ref[t // tpc2, :, sl:sl + EMB]
                 + p_ref[:, t // tpc2, sl:sl + EMB]) * dv + add_t
            g = (jnp.dot(e, wih, preferred_element_type=jnp.float32)
                 + jnp.dot(h, whh, preferred_element_type=jnp.float32) + bias)
            i_g = jax.nn.sigmoid(g[:, 0:H])
            f_g = jax.nn.sigmoid(g[:, H:2 * H])
            g_g = jnp.tanh(g[:, 2 * H:3 * H])
            o_g = jax.nn.sigmoid(g[:, 3 * H:4 * H])
            c = f_g * c + i_g * g_g
            h = o_g * jnp.tanh(c)
        z2 = jnp.maximum(
            jnp.dot(h, wm1_ref[...], preferred_element_type=jnp.float32)
            + bm1_ref[...], 0.0)
        pred = (jnp.dot(z2, wm2_ref[...], preferred_element_type=jnp.float32)
                + bm2_ref[...])
        o_ref[...] = pred[:, 0]

    return pl.pallas_call(
        kfn,
        grid=(NP // BN,),
        in_specs=[pl.BlockSpec((NCH2, BN, FC), lambda i: (0, i, 0)),
                  pl.BlockSpec((BN, NCH2, FC), lambda i: (i, 0, 0)),
                  pl.BlockSpec((BN,), lambda i: (i,)),
                  pl.BlockSpec((BN, EMB), lambda i: (i, 0)),
                  pl.BlockSpec((1, EMB), lambda i: (0, 0)),
                  pl.BlockSpec((EMB, 4 * H), lambda i: (0, 0)),
                  pl.BlockSpec((H, 4 * H), lambda i: (0, 0)),
                  pl.BlockSpec((1, 4 * H), lambda i: (0, 0)),
                  pl.BlockSpec((1, 4 * H), lambda i: (0, 0)),
                  pl.BlockSpec((H, H // 2), lambda i: (0, 0)),
                  pl.BlockSpec((1, H // 2), lambda i: (0, 0)),
                  pl.BlockSpec((H // 2, 1), lambda i: (0, 0)),
                  pl.BlockSpec((1, 1), lambda i: (0, 0))],
        out_specs=pl.BlockSpec((BN,), lambda i: (i,)),
        out_shape=jax.ShapeDtypeStruct((NP,), jnp.float32),
    )(Z, t2s, dinv, cbp, b2r, WihT, WhhT, bihr, bhhr, Wm1T, bm1r, Wm2T, bm2r)


def kernel(x, edge_index, edge_weight, W1, b1, W2, b2, county_bias,
           W_ih, W_hh, b_ih, b_hh, Wm1, bm1, Wm2, bm2):
    rows = edge_index[0].astype(jnp.int32)
    cols = edge_index[1].astype(jnp.int32)
    w = edge_weight.astype(jnp.float32)

    xp = jnp.pad(x, ((0, 0), (0, NP - N), (0, FPAD - FEAT)))
    Xt = xp.transpose(1, 0, 2).reshape(NP, NCH1, FC)
    z1 = jnp.zeros((NP,), jnp.float32)
    z2 = jnp.zeros((RPT, FC), jnp.float32)

    pdeg, packed = _sc_deg(rows, cols, w, z1)
    dinv, t1s = _tc_prep(pdeg, Xt)

    tab1 = t1s.reshape(NP * NCH1, FC)
    Y = _sc_spmm(tab1, packed, w, z2, NCH1)

    W1p = jnp.pad(W1, ((0, FPAD - FEAT), (0, 0)))
    t2s = _tc_mid(Y, t1s, dinv, W1p, b1[None], W2)

    tab2 = t2s.reshape(NP * NCH2, FC)
    Z = _sc_spmm(tab2, packed, w, z2, NCH2)

    cbp = jnp.pad(county_bias, ((0, NP - N), (0, 0)))
    preds = _tc_final(Z, t2s, dinv, cbp, b2[None], W_ih.T, W_hh.T,
                      b_ih[None], b_hh[None], Wm1.T, bm1[None], Wm2.T,
                      bm2[None])
    return preds[:N]


# speedup vs baseline: 20.8500x; 1.2426x over previous
import functools

import jax
import jax.numpy as jnp
from jax import lax
from jax.experimental import pallas as pl
from jax.experimental.pallas import tpu as pltpu
from jax.experimental.pallas import tpu_sc as plsc

N = 10000
NP = 10240
E = 320000
T = 24
FEAT = 15
FPAD = 16
EMB = 64
H = 128
FC = 128
NCH1 = 3
NCH2 = 12
EB = 80
BB = 50
NACC = 10112
BN = 512
RPT = NACC // 16
PK = 16384


def _sc_deg(rows, cols, w, zeros1d):
    ept = E // 32
    nb = ept // EB
    mesh = plsc.VectorSubcoreMesh(core_axis_name="c", subcore_axis_name="s")

    @functools.partial(
        pl.kernel, mesh=mesh,
        out_type=[jax.ShapeDtypeStruct((32, NP), jnp.float32),
                  jax.ShapeDtypeStruct((E,), jnp.int32)],
        compiler_params=pltpu.CompilerParams(needs_layout_passes=False),
        scratch_types=[
            pltpu.VMEM((EB,), jnp.int32),
            pltpu.VMEM((EB,), jnp.int32),
            pltpu.VMEM((EB,), jnp.float32),
            pltpu.VMEM((EB,), jnp.int32),
            pltpu.VMEM((NP,), jnp.float32),
        ],
    )
    def k(rows_hbm, cols_hbm, w_hbm, z_hbm, out_hbm, pk_hbm,
          rowb, colb, wb, packb, deg_l):
        cc = lax.axis_index("c")
        ss = lax.axis_index("s")
        wid = ss * 2 + cc
        base = wid * ept
        pltpu.sync_copy(z_hbm, deg_l)

        def body(b, carry):
            start = base + b * EB
            pltpu.sync_copy(rows_hbm.at[pl.ds(start, EB)], rowb)
            pltpu.sync_copy(cols_hbm.at[pl.ds(start, EB)], colb)
            pltpu.sync_copy(w_hbm.at[pl.ds(start, EB)], wb)
            for g in range(EB // 16):
                ci = colb[pl.ds(g * 16, 16)]
                wv = wb[pl.ds(g * 16, 16)]
                plsc.addupdate_scatter(deg_l, [ci], wv)
                rv = rowb[pl.ds(g * 16, 16)]
                packb[pl.ds(g * 16, 16)] = rv * PK + ci
            pltpu.sync_copy(packb, pk_hbm.at[pl.ds(start, EB)])
            return carry

        lax.fori_loop(0, nb, body, 0)
        pltpu.sync_copy(deg_l, out_hbm.at[wid])

    return k(rows, cols, w, zeros1d)


def _sc_spmm(table, packed, w, zeros2d, nch):
    n0 = (nch + 1) // 2
    ept = E // 16
    nbat = ept // EB
    nblk = nbat // BB
    bedg = BB * EB
    mesh = plsc.VectorSubcoreMesh(core_axis_name="c", subcore_axis_name="s")

    @functools.partial(
        pl.kernel, mesh=mesh,
        out_type=jax.ShapeDtypeStruct((nch, NP, FC), jnp.float32),
        compiler_params=pltpu.CompilerParams(needs_layout_passes=False),
        scratch_types=[
            pltpu.VMEM((bedg,), jnp.int32),
            pltpu.VMEM((bedg,), jnp.float32),
            pltpu.VMEM((EB,), jnp.int32),
            pltpu.VMEM((EB,), jnp.int32),
            pltpu.VMEM((EB,), jnp.int32),
            pltpu.VMEM((EB,), jnp.int32),
            pltpu.VMEM((EB, FC), jnp.float32),
            pltpu.VMEM((EB, FC), jnp.float32),
            pltpu.VMEM((EB, FC), jnp.float32),
            pltpu.VMEM((EB, FC), jnp.float32),
            pltpu.VMEM_SHARED((NACC, FC), jnp.float32),
            pltpu.SemaphoreType.DMA,
            pltpu.SemaphoreType.DMA,
            pltpu.SemaphoreType.DMA,
            pltpu.SemaphoreType.DMA,
        ],
    )
    def k(tab_hbm, pk_hbm, w_hbm, z_hbm, out_hbm,
          pk_blk, w_blk, idxb0, idxb1, colb0, colb1,
          buf0, buf1, sbuf0, sbuf1, acc, sem0, sem1, semS0, semS1):
        cc = lax.axis_index("c")
        ss = lax.axis_index("s")

        def make_rows(lb, idxb, chunk_g):
            for g in range(EB // 16):
                pv = pk_blk[pl.ds(lb * EB + g * 16, 16)]
                rv = lax.shift_right_logical(pv, 14)
                idxb[pl.ds(g * 16, 16)] = rv * nch + chunk_g

        def make_cols(lb, colb):
            for g in range(EB // 16):
                pv = pk_blk[pl.ds(lb * EB + g * 16, 16)]
                colb[pl.ds(g * 16, 16)] = lax.bitwise_and(pv, PK - 1)

        def scale(lb, buf, sbuf):
            for e in range(EB):
                wsp = plsc.load_gather(
                    w_blk, [jnp.full((16,), lb * EB + e, jnp.int32)])
                for kk in range(FC // 16):
                    v = buf[e, pl.ds(kk * 16, 16)]
                    sbuf[e, pl.ds(kk * 16, 16)] = v * wsp

        def chunk_body(j, carry0):
            chunk_g = cc * n0 + j
            pltpu.sync_copy(z_hbm, acc.at[pl.ds(ss * RPT, RPT)])
            plsc.subcore_barrier()

            def block_body(blk, carry1):
                ebase = ss * ept + blk * bedg
                pltpu.sync_copy(pk_hbm.at[pl.ds(ebase, bedg)], pk_blk)
                pltpu.sync_copy(w_hbm.at[pl.ds(ebase, bedg)], w_blk)
                make_rows(0, idxb0, chunk_g)
                pltpu.async_copy(tab_hbm.at[idxb0], buf0, sem0)
                make_rows(1, idxb1, chunk_g)
                pltpu.async_copy(tab_hbm.at[idxb1], buf1, sem1)

                def body(i, carry):
                    a = 2 * i

                    @pl.when(i > 0)
                    def _():
                        pltpu.make_async_copy(sbuf0, acc.at[colb0],
                                              semS0).wait()
                    pltpu.make_async_copy(tab_hbm.at[idxb0], buf0, sem0).wait()
                    scale(a, buf0, sbuf0)
                    make_rows(lax.rem(a + 2, BB), idxb0, chunk_g)
                    pltpu.async_copy(tab_hbm.at[idxb0], buf0, sem0)
                    make_cols(a, colb0)
                    pltpu.async_copy(sbuf0, acc.at[colb0], semS0, add=True)

                    @pl.when(i > 0)
                    def _():
                        pltpu.make_async_copy(sbuf1, acc.at[colb1],
                                              semS1).wait()
                    pltpu.make_async_copy(tab_hbm.at[idxb1], buf1, sem1).wait()
                    scale(a + 1, buf1, sbuf1)
                    make_rows(lax.rem(a + 3, BB), idxb1, chunk_g)
                    pltpu.async_copy(tab_hbm.at[idxb1], buf1, sem1)
                    make_cols(a + 1, colb1)
                    pltpu.async_copy(sbuf1, acc.at[colb1], semS1, add=True)
                    return carry

                lax.fori_loop(0, BB // 2, body, 0)
                pltpu.make_async_copy(sbuf0, acc.at[colb0], semS0).wait()
                pltpu.make_async_copy(sbuf1, acc.at[colb1], semS1).wait()
                pltpu.make_async_copy(tab_hbm.at[idxb0], buf0, sem0).wait()
                pltpu.make_async_copy(tab_hbm.at[idxb1], buf1, sem1).wait()
                return carry1

            lax.fori_loop(0, nblk, block_body, 0)
            plsc.subcore_barrier()
            pltpu.sync_copy(acc.at[pl.ds(ss * RPT, RPT)],
                            out_hbm.at[chunk_g, pl.ds(ss * RPT, RPT)])
            return carry0

        nloc = jnp.where(cc == 0, n0, nch - n0)
        lax.fori_loop(0, nloc, chunk_body, 0)

    return k(table, packed, w, zeros2d)


def _tc_prep(pdeg, Xt):
    def kfn(pd_ref, xt_ref, dinv_ref, t1_ref):
        d = jnp.sum(pd_ref[...], axis=0) + 1.0
        dv = lax.rsqrt(d)
        dinv_ref[...] = dv
        t1_ref[...] = xt_ref[...] * dv[:, None, None]

    return pl.pallas_call(
        kfn,
        grid=(NP // BN,),
        in_specs=[pl.BlockSpec((32, BN), lambda i: (0, i)),
                  pl.BlockSpec((BN, NCH1, FC), lambda i: (i, 0, 0))],
        out_specs=[pl.BlockSpec((BN,), lambda i: (i,)),
                   pl.BlockSpec((BN, NCH1, FC), lambda i: (i, 0, 0))],
        out_shape=[jax.ShapeDtypeStruct((NP,), jnp.float32),
                   jax.ShapeDtypeStruct((NP, NCH1, FC), jnp.float32)],
    )(pdeg, Xt)


def _tc_mid(Y, t1s, dinv, W1p, b1r, W2):
    def kfn(y_ref, x_ref, dv_ref, w1_ref, b1_ref, w2_ref, o_ref):
        dv = dv_ref[...][:, None]
        w1 = w1_ref[...]
        b1v = b1_ref[...]
        w2 = w2_ref[...]
        tpc1 = FC // FPAD
        tpc2 = FC // EMB
        for t in range(T):
            c1, o1 = t // tpc1, (t % tpc1) * FPAD
            ax = (y_ref[c1, :, o1:o1 + FPAD] + x_ref[:, c1, o1:o1 + FPAD]) * dv
            h1 = jnp.maximum(
                jnp.dot(ax, w1, preferred_element_type=jnp.float32) + b1v, 0.0)
            p = jnp.dot(h1, w2, preferred_element_type=jnp.float32)
            o2 = (t % tpc2) * EMB
            o_ref[:, t // tpc2, o2:o2 + EMB] = p * dv

    return pl.pallas_call(
        kfn,
        grid=(NP // BN,),
        in_specs=[pl.BlockSpec((NCH1, BN, FC), lambda i: (0, i, 0)),
                  pl.BlockSpec((BN, NCH1, FC), lambda i: (i, 0, 0)),
                  pl.BlockSpec((BN,), lambda i: (i,)),
                  pl.BlockSpec((FPAD, EMB), lambda i: (0, 0)),
                  pl.BlockSpec((1, EMB), lambda i: (0, 0)),
                  pl.BlockSpec((EMB, EMB), lambda i: (0, 0))],
        out_specs=pl.BlockSpec((BN, NCH2, FC), lambda i: (i, 0, 0)),
        out_shape=jax.ShapeDtypeStruct((NP, NCH2, FC), jnp.float32),
    )(Y, t1s, dinv, W1p, b1r, W2)


def _tc_final(Z, t2s, dinv, cbp, b2r, WihT, WhhT, bihr, bhhr,
              Wm1T, bm1r, Wm2T, bm2r):
    def kfn(z_ref, p_ref, dv_ref, cb_ref, b2_ref, wih_ref, whh_ref,
            bi_ref, bh_ref, wm1_ref, bm1_ref, wm2_ref, bm2_ref, o_ref):
        dv = dv_ref[...][:, None]
        add_t = b2_ref[...] + cb_ref[...]
        bias = bi_ref[...] + bh_ref[...]
        wih = wih_ref[...]
        whh = whh_ref[...]
        h = jnp.zeros((BN, H), jnp.float32)
        c = jnp.zeros((BN, H), jnp.float32)
        tpc2 = FC // EMB
        for t in range(T):
            sl = (t % tpc2) * EMB
            e = (z_ref[t // tpc2, :, sl:sl + EMB]
                 + p_ref[:, t // tpc2, sl:sl + EMB]) * dv + add_t
            g = (jnp.dot(e, wih, preferred_element_type=jnp.float32)
                 + jnp.dot(h, whh, preferred_element_type=jnp.float32) + bias)
            i_g = jax.nn.sigmoid(g[:, 0:H])
            f_g = jax.nn.sigmoid(g[:, H:2 * H])
            g_g = jnp.tanh(g[:, 2 * H:3 * H])
            o_g = jax.nn.sigmoid(g[:, 3 * H:4 * H])
            c = f_g * c + i_g * g_g
            h = o_g * jnp.tanh(c)
        z2 = jnp.maximum(
            jnp.dot(h, wm1_ref[...], preferred_element_type=jnp.float32)
            + bm1_ref[...], 0.0)
        pred = (jnp.dot(z2, wm2_ref[...], preferred_element_type=jnp.float32)
                + bm2_ref[...])
        o_ref[...] = pred[:, 0]

    return pl.pallas_call(
        kfn,
        grid=(NP // BN,),
        in_specs=[pl.BlockSpec((NCH2, BN, FC), lambda i: (0, i, 0)),
                  pl.BlockSpec((BN, NCH2, FC), lambda i: (i, 0, 0)),
                  pl.BlockSpec((BN,), lambda i: (i,)),
                  pl.BlockSpec((BN, EMB), lambda i: (i, 0)),
                  pl.BlockSpec((1, EMB), lambda i: (0, 0)),
                  pl.BlockSpec((EMB, 4 * H), lambda i: (0, 0)),
                  pl.BlockSpec((H, 4 * H), lambda i: (0, 0)),
                  pl.BlockSpec((1, 4 * H), lambda i: (0, 0)),
                  pl.BlockSpec((1, 4 * H), lambda i: (0, 0)),
                  pl.BlockSpec((H, H // 2), lambda i: (0, 0)),
                  pl.BlockSpec((1, H // 2), lambda i: (0, 0)),
                  pl.BlockSpec((H // 2, 1), lambda i: (0, 0)),
                  pl.BlockSpec((1, 1), lambda i: (0, 0))],
        out_specs=pl.BlockSpec((BN,), lambda i: (i,)),
        out_shape=jax.ShapeDtypeStruct((NP,), jnp.float32),
    )(Z, t2s, dinv, cbp, b2r, WihT, WhhT, bihr, bhhr, Wm1T, bm1r, Wm2T, bm2r)


def kernel(x, edge_index, edge_weight, W1, b1, W2, b2, county_bias,
           W_ih, W_hh, b_ih, b_hh, Wm1, bm1, Wm2, bm2):
    rows = edge_index[0].astype(jnp.int32)
    cols = edge_index[1].astype(jnp.int32)
    w = edge_weight.astype(jnp.float32)

    xp = jnp.pad(x, ((0, 0), (0, NP - N), (0, FPAD - FEAT)))
    Xt = xp.transpose(1, 0, 2).reshape(NP, NCH1, FC)
    z1 = jnp.zeros((NP,), jnp.float32)
    z2 = jnp.zeros((RPT, FC), jnp.float32)

    pdeg, packed = _sc_deg(rows, cols, w, z1)
    dinv, t1s = _tc_prep(pdeg, Xt)

    tab1 = t1s.reshape(NP * NCH1, FC)
    Y = _sc_spmm(tab1, packed, w, z2, NCH1)

    W1p = jnp.pad(W1, ((0, FPAD - FEAT), (0, 0)))
    t2s = _tc_mid(Y, t1s, dinv, W1p, b1[None], W2)

    tab2 = t2s.reshape(NP * NCH2, FC)
    Z = _sc_spmm(tab2, packed, w, z2, NCH2)

    cbp = jnp.pad(county_bias, ((0, NP - N), (0, 0)))
    preds = _tc_final(Z, t2s, dinv, cbp, b2[None], W_ih.T, W_hh.T,
                      b_ih[None], b_hh[None], Wm1.T, bm1[None], Wm2.T,
                      bm2[None])
    return preds[:N]


# speedup vs baseline: 21.8563x; 1.0483x over previous
import functools

import jax
import jax.numpy as jnp
from jax import lax
from jax.experimental import pallas as pl
from jax.experimental.pallas import tpu as pltpu
from jax.experimental.pallas import tpu_sc as plsc

N = 10000
NP = 10240
E = 320000
T = 24
FEAT = 15
FPAD = 16
EMB = 64
H = 128
FC = 128
NCH1 = 3
NCH2 = 12
EB = 80
BB = 50
NACC = 10112
BN = 512
RPT = NACC // 16
PK = 16384


def _sc_deg(rows, cols, w, zeros1d):
    ept = E // 32
    nb = ept // EB
    mesh = plsc.VectorSubcoreMesh(core_axis_name="c", subcore_axis_name="s")

    @functools.partial(
        pl.kernel, mesh=mesh,
        out_type=[jax.ShapeDtypeStruct((32, NP), jnp.float32),
                  jax.ShapeDtypeStruct((E,), jnp.int32)],
        compiler_params=pltpu.CompilerParams(needs_layout_passes=False),
        scratch_types=[
            pltpu.VMEM((EB,), jnp.int32),
            pltpu.VMEM((EB,), jnp.int32),
            pltpu.VMEM((EB,), jnp.float32),
            pltpu.VMEM((EB,), jnp.int32),
            pltpu.VMEM((NP,), jnp.float32),
        ],
    )
    def k(rows_hbm, cols_hbm, w_hbm, z_hbm, out_hbm, pk_hbm,
          rowb, colb, wb, packb, deg_l):
        cc = lax.axis_index("c")
        ss = lax.axis_index("s")
        wid = ss * 2 + cc
        base = wid * ept
        pltpu.sync_copy(z_hbm, deg_l)

        def body(b, carry):
            start = base + b * EB
            pltpu.sync_copy(rows_hbm.at[pl.ds(start, EB)], rowb)
            pltpu.sync_copy(cols_hbm.at[pl.ds(start, EB)], colb)
            pltpu.sync_copy(w_hbm.at[pl.ds(start, EB)], wb)
            for g in range(EB // 16):
                ci = colb[pl.ds(g * 16, 16)]
                wv = wb[pl.ds(g * 16, 16)]
                plsc.addupdate_scatter(deg_l, [ci], wv)
                rv = rowb[pl.ds(g * 16, 16)]
                packb[pl.ds(g * 16, 16)] = rv * PK + ci
            pltpu.sync_copy(packb, pk_hbm.at[pl.ds(start, EB)])
            return carry

        lax.fori_loop(0, nb, body, 0)
        pltpu.sync_copy(deg_l, out_hbm.at[wid])

    return k(rows, cols, w, zeros1d)


def _sc_spmm(table, packed, w, zeros2d, nch):
    n0 = (nch + 1) // 2
    ept = E // 16
    nbat = ept // EB
    bedg = BB * EB
    half0 = (nbat // 2) & ~1
    nout = nch + (nch % 2)
    mesh = plsc.VectorSubcoreMesh(core_axis_name="c", subcore_axis_name="s")

    @functools.partial(
        pl.kernel, mesh=mesh,
        out_type=jax.ShapeDtypeStruct((nout, NP, FC), jnp.float32),
        compiler_params=pltpu.CompilerParams(needs_layout_passes=False),
        scratch_types=[
            pltpu.VMEM((bedg,), jnp.int32),
            pltpu.VMEM((bedg,), jnp.float32),
            pltpu.VMEM((EB,), jnp.int32),
            pltpu.VMEM((EB,), jnp.int32),
            pltpu.VMEM((EB,), jnp.int32),
            pltpu.VMEM((EB,), jnp.int32),
            pltpu.VMEM((EB, FC), jnp.float32),
            pltpu.VMEM((EB, FC), jnp.float32),
            pltpu.VMEM((EB, FC), jnp.float32),
            pltpu.VMEM((EB, FC), jnp.float32),
            pltpu.VMEM_SHARED((NACC, FC), jnp.float32),
            pltpu.SemaphoreType.DMA,
            pltpu.SemaphoreType.DMA,
            pltpu.SemaphoreType.DMA,
            pltpu.SemaphoreType.DMA,
        ],
    )
    def k(tab_hbm, pk_hbm, w_hbm, z_hbm, out_hbm,
          pk_blk, w_blk, idxb0, idxb1, colb0, colb1,
          buf0, buf1, sbuf0, sbuf1, acc, sem0, sem1, semS0, semS1):
        cc = lax.axis_index("c")
        ss = lax.axis_index("s")

        def make_rows(lb, idxb, chunk_g):
            for g in range(EB // 16):
                pv = pk_blk[pl.ds(lb * EB + g * 16, 16)]
                rv = lax.shift_right_logical(pv, 14)
                idxb[pl.ds(g * 16, 16)] = rv * nch + chunk_g

        def make_cols(lb, colb):
            for g in range(EB // 16):
                pv = pk_blk[pl.ds(lb * EB + g * 16, 16)]
                colb[pl.ds(g * 16, 16)] = lax.bitwise_and(pv, PK - 1)

        def scale(lb, buf, sbuf):
            for e in range(EB):
                wsp = plsc.load_gather(
                    w_blk, [jnp.full((16,), lb * EB + e, jnp.int32)])
                for kk in range(FC // 16):
                    v = buf[e, pl.ds(kk * 16, 16)]
                    sbuf[e, pl.ds(kk * 16, 16)] = v * wsp

        def run_chunk(chunk_g, out_row, ebase0, nbat_j):
            pltpu.sync_copy(z_hbm, acc.at[pl.ds(ss * RPT, RPT)])
            plsc.subcore_barrier()

            def block_body(blk, carry1):
                bstart = blk * BB
                bcount = jnp.minimum(BB, nbat_j - bstart)
                pltpu.sync_copy(pk_hbm.at[pl.ds(ebase0 + bstart * EB, bedg)],
                                pk_blk)
                pltpu.sync_copy(w_hbm.at[pl.ds(ebase0 + bstart * EB, bedg)],
                                w_blk)
                make_rows(0, idxb0, chunk_g)
                pltpu.async_copy(tab_hbm.at[idxb0], buf0, sem0)
                make_rows(1, idxb1, chunk_g)
                pltpu.async_copy(tab_hbm.at[idxb1], buf1, sem1)

                def body(i, carry):
                    a = 2 * i

                    @pl.when(i > 0)
                    def _():
                        pltpu.make_async_copy(sbuf0, acc.at[colb0],
                                              semS0).wait()
                    pltpu.make_async_copy(tab_hbm.at[idxb0], buf0, sem0).wait()
                    scale(a, buf0, sbuf0)
                    make_rows(lax.rem(a + 2, bcount), idxb0, chunk_g)
                    pltpu.async_copy(tab_hbm.at[idxb0], buf0, sem0)
                    make_cols(a, colb0)
                    pltpu.async_copy(sbuf0, acc.at[colb0], semS0, add=True)

                    @pl.when(i > 0)
                    def _():
                        pltpu.make_async_copy(sbuf1, acc.at[colb1],
                                              semS1).wait()
                    pltpu.make_async_copy(tab_hbm.at[idxb1], buf1, sem1).wait()
                    scale(a + 1, buf1, sbuf1)
                    make_rows(lax.rem(a + 3, bcount), idxb1, chunk_g)
                    pltpu.async_copy(tab_hbm.at[idxb1], buf1, sem1)
                    make_cols(a + 1, colb1)
                    pltpu.async_copy(sbuf1, acc.at[colb1], semS1, add=True)
                    return carry

                lax.fori_loop(0, bcount // 2, body, 0)
                pltpu.make_async_copy(sbuf0, acc.at[colb0], semS0).wait()
                pltpu.make_async_copy(sbuf1, acc.at[colb1], semS1).wait()
                pltpu.make_async_copy(tab_hbm.at[idxb0], buf0, sem0).wait()
                pltpu.make_async_copy(tab_hbm.at[idxb1], buf1, sem1).wait()
                return carry1

            lax.fori_loop(0, (nbat_j + BB - 1) // BB, block_body, 0)
            plsc.subcore_barrier()
            pltpu.sync_copy(acc.at[pl.ds(ss * RPT, RPT)],
                            out_hbm.at[out_row, pl.ds(ss * RPT, RPT)])

        if nch % 2 == 0:
            def chunk_body(j, carry0):
                chunk_g = cc * n0 + j
                run_chunk(chunk_g, chunk_g, ss * ept, nbat)
                return carry0

            lax.fori_loop(0, n0, chunk_body, 0)
        else:
            def chunk_body(j, carry0):
                split = j == n0 - 1
                chunk_g = jnp.where(split, nch - 1, cc * (n0 - 1) + j)
                out_row = jnp.where(split, nch - 1 + cc, chunk_g)
                ebase0 = ss * ept + jnp.where(split & (cc == 1),
                                              half0 * EB, 0)
                nbat_j = jnp.where(split,
                                   jnp.where(cc == 0, half0, nbat - half0),
                                   nbat)
                run_chunk(chunk_g, out_row, ebase0, nbat_j)
                return carry0

            lax.fori_loop(0, n0, chunk_body, 0)

    return k(table, packed, w, zeros2d)


def _tc_prep(pdeg, Xt):
    def kfn(pd_ref, xt_ref, dinv_ref, t1_ref):
        d = jnp.sum(pd_ref[...], axis=0) + 1.0
        dv = lax.rsqrt(d)
        dinv_ref[...] = dv
        t1_ref[...] = xt_ref[...] * dv[:, None, None]

    return pl.pallas_call(
        kfn,
        grid=(NP // BN,),
        in_specs=[pl.BlockSpec((32, BN), lambda i: (0, i)),
                  pl.BlockSpec((BN, NCH1, FC), lambda i: (i, 0, 0))],
        out_specs=[pl.BlockSpec((BN,), lambda i: (i,)),
                   pl.BlockSpec((BN, NCH1, FC), lambda i: (i, 0, 0))],
        out_shape=[jax.ShapeDtypeStruct((NP,), jnp.float32),
                   jax.ShapeDtypeStruct((NP, NCH1, FC), jnp.float32)],
    )(pdeg, Xt)


def _tc_mid(Y, t1s, dinv, W1p, b1r, W2):
    def kfn(y_ref, x_ref, dv_ref, w1_ref, b1_ref, w2_ref, o_ref):
        dv = dv_ref[...][:, None]
        w1 = w1_ref[...]
        b1v = b1_ref[...]
        w2 = w2_ref[...]
        tpc1 = FC // FPAD
        tpc2 = FC // EMB
        for t in range(T):
            c1, o1 = t // tpc1, (t % tpc1) * FPAD
            if c1 < NCH1 - 1:
                ysl = y_ref[c1, :, o1:o1 + FPAD]
            else:
                ysl = (y_ref[NCH1 - 1, :, o1:o1 + FPAD]
                       + y_ref[NCH1, :, o1:o1 + FPAD])
            ax = (ysl + x_ref[:, c1, o1:o1 + FPAD]) * dv
            h1 = jnp.maximum(
                jnp.dot(ax, w1, preferred_element_type=jnp.float32) + b1v, 0.0)
            p = jnp.dot(h1, w2, preferred_element_type=jnp.float32)
            o2 = (t % tpc2) * EMB
            o_ref[:, t // tpc2, o2:o2 + EMB] = p * dv

    return pl.pallas_call(
        kfn,
        grid=(NP // BN,),
        in_specs=[pl.BlockSpec((NCH1 + 1, BN, FC), lambda i: (0, i, 0)),
                  pl.BlockSpec((BN, NCH1, FC), lambda i: (i, 0, 0)),
                  pl.BlockSpec((BN,), lambda i: (i,)),
                  pl.BlockSpec((FPAD, EMB), lambda i: (0, 0)),
                  pl.BlockSpec((1, EMB), lambda i: (0, 0)),
                  pl.BlockSpec((EMB, EMB), lambda i: (0, 0))],
        out_specs=pl.BlockSpec((BN, NCH2, FC), lambda i: (i, 0, 0)),
        out_shape=jax.ShapeDtypeStruct((NP, NCH2, FC), jnp.float32),
    )(Y, t1s, dinv, W1p, b1r, W2)


def _tc_final(Z, t2s, dinv, cbp, b2r, WihT, WhhT, bihr, bhhr,
              Wm1T, bm1r, Wm2T, bm2r):
    def kfn(z_ref, p_ref, dv_ref, cb_ref, b2_ref, wih_ref, whh_ref,
            bi_ref, bh_ref, wm1_ref, bm1_ref, wm2_ref, bm2_ref, o_ref):
        dv = dv_ref[...][:, None]
        add_t = b2_ref[...] + cb_ref[...]
        bias = bi_ref[...] + bh_ref[...]
        wih = wih_ref[...]
        whh = whh_ref[...]
        h = jnp.zeros((BN, H), jnp.float32)
        c = jnp.zeros((BN, H), jnp.float32)
        tpc2 = FC // EMB
        for t in range(T):
            sl = (t % tpc2) * EMB
            e = (z_ref[t // tpc2, :, sl:sl + EMB]
                 + p_ref[:, t // tpc2, sl:sl + EMB]) * dv + add_t
            g = (jnp.dot(e, wih, preferred_element_type=jnp.float32)
                 + jnp.dot(h, whh, preferred_element_type=jnp.float32) + bias)
            i_g = jax.nn.sigmoid(g[:, 0:H])
            f_g = jax.nn.sigmoid(g[:, H:2 * H])
            g_g = jnp.tanh(g[:, 2 * H:3 * H])
            o_g = jax.nn.sigmoid(g[:, 3 * H:4 * H])
            c = f_g * c + i_g * g_g
            h = o_g * jnp.tanh(c)
        z2 = jnp.maximum(
            jnp.dot(h, wm1_ref[...], preferred_element_type=jnp.float32)
            + bm1_ref[...], 0.0)
        pred = (jnp.dot(z2, wm2_ref[...], preferred_element_type=jnp.float32)
                + bm2_ref[...])
        o_ref[...] = pred[:, 0]

    return pl.pallas_call(
        kfn,
        grid=(NP // BN,),
        in_specs=[pl.BlockSpec((NCH2, BN, FC), lambda i: (0, i, 0)),
                  pl.BlockSpec((BN, NCH2, FC), lambda i: (i, 0, 0)),
                  pl.BlockSpec((BN,), lambda i: (i,)),
                  pl.BlockSpec((BN, EMB), lambda i: (i, 0)),
                  pl.BlockSpec((1, EMB), lambda i: (0, 0)),
                  pl.BlockSpec((EMB, 4 * H), lambda i: (0, 0)),
                  pl.BlockSpec((H, 4 * H), lambda i: (0, 0)),
                  pl.BlockSpec((1, 4 * H), lambda i: (0, 0)),
                  pl.BlockSpec((1, 4 * H), lambda i: (0, 0)),
                  pl.BlockSpec((H, H // 2), lambda i: (0, 0)),
                  pl.BlockSpec((1, H // 2), lambda i: (0, 0)),
                  pl.BlockSpec((H // 2, 1), lambda i: (0, 0)),
                  pl.BlockSpec((1, 1), lambda i: (0, 0))],
        out_specs=pl.BlockSpec((BN,), lambda i: (i,)),
        out_shape=jax.ShapeDtypeStruct((NP,), jnp.float32),
    )(Z, t2s, dinv, cbp, b2r, WihT, WhhT, bihr, bhhr, Wm1T, bm1r, Wm2T, bm2r)


def kernel(x, edge_index, edge_weight, W1, b1, W2, b2, county_bias,
           W_ih, W_hh, b_ih, b_hh, Wm1, bm1, Wm2, bm2):
    rows = edge_index[0].astype(jnp.int32)
    cols = edge_index[1].astype(jnp.int32)
    w = edge_weight.astype(jnp.float32)

    xp = jnp.pad(x, ((0, 0), (0, NP - N), (0, FPAD - FEAT)))
    Xt = xp.transpose(1, 0, 2).reshape(NP, NCH1, FC)
    z1 = jnp.zeros((NP,), jnp.float32)
    z2 = jnp.zeros((RPT, FC), jnp.float32)

    pdeg, packed = _sc_deg(rows, cols, w, z1)
    packed_p = jnp.pad(packed, (0, BB * EB))
    w_p = jnp.pad(w, (0, BB * EB))
    dinv, t1s = _tc_prep(pdeg, Xt)

    tab1 = t1s.reshape(NP * NCH1, FC)
    Y = _sc_spmm(tab1, packed_p, w_p, z2, NCH1)

    W1p = jnp.pad(W1, ((0, FPAD - FEAT), (0, 0)))
    t2s = _tc_mid(Y, t1s, dinv, W1p, b1[None], W2)

    tab2 = t2s.reshape(NP * NCH2, FC)
    Z = _sc_spmm(tab2, packed_p, w_p, z2, NCH2)

    cbp = jnp.pad(county_bias, ((0, NP - N), (0, 0)))
    preds = _tc_final(Z, t2s, dinv, cbp, b2[None], W_ih.T, W_hh.T,
                      b_ih[None], b_hh[None], Wm1.T, bm1[None], Wm2.T,
                      bm2[None])
    return preds[:N]


# speedup vs baseline: 22.5864x; 1.0334x over previous
import functools

import jax
import jax.numpy as jnp
from jax import lax
from jax.experimental import pallas as pl
from jax.experimental.pallas import tpu as pltpu
from jax.experimental.pallas import tpu_sc as plsc

N = 10000
NP = 10240
E = 320000
T = 24
FEAT = 15
FPAD = 16
EMB = 64
H = 128
FC = 128
NCH1 = 3
NCH2 = 12
EB = 80
BB = 50
NACC = 10112
BN = 512
RPT = NACC // 16
PK = 16384


def _sc_deg(rows, cols, w, zeros1d):
    ept = E // 32
    ebd = 2000
    nst = ept // ebd
    mesh = plsc.VectorSubcoreMesh(core_axis_name="c", subcore_axis_name="s")

    @functools.partial(
        pl.kernel, mesh=mesh,
        out_type=[jax.ShapeDtypeStruct((32, NP), jnp.float32),
                  jax.ShapeDtypeStruct((E,), jnp.int32)],
        compiler_params=pltpu.CompilerParams(needs_layout_passes=False),
        scratch_types=[
            pltpu.VMEM((ebd,), jnp.int32),
            pltpu.VMEM((ebd,), jnp.int32),
            pltpu.VMEM((ebd,), jnp.float32),
            pltpu.VMEM((ebd,), jnp.int32),
            pltpu.VMEM((NP,), jnp.float32),
        ],
    )
    def k(rows_hbm, cols_hbm, w_hbm, z_hbm, out_hbm, pk_hbm,
          rowb, colb, wb, packb, deg_l):
        cc = lax.axis_index("c")
        ss = lax.axis_index("s")
        wid = ss * 2 + cc
        base = wid * ept
        pltpu.sync_copy(z_hbm, deg_l)

        def body(b, carry):
            start = base + b * ebd
            pltpu.sync_copy(rows_hbm.at[pl.ds(start, ebd)], rowb)
            pltpu.sync_copy(cols_hbm.at[pl.ds(start, ebd)], colb)
            pltpu.sync_copy(w_hbm.at[pl.ds(start, ebd)], wb)
            for g in range(ebd // 16):
                ci = colb[pl.ds(g * 16, 16)]
                wv = wb[pl.ds(g * 16, 16)]
                plsc.addupdate_scatter(deg_l, [ci], wv)
                rv = rowb[pl.ds(g * 16, 16)]
                packb[pl.ds(g * 16, 16)] = rv * PK + ci
            pltpu.sync_copy(packb, pk_hbm.at[pl.ds(start, ebd)])
            return carry

        lax.fori_loop(0, nst, body, 0)
        pltpu.sync_copy(deg_l, out_hbm.at[wid])

    return k(rows, cols, w, zeros1d)


def _sc_spmm(table, packed, w, zeros2d, nch):
    n0 = (nch + 1) // 2
    ept = E // 16
    nbat = ept // EB
    bedg = BB * EB
    half0 = (nbat // 2) & ~1
    nout = nch + (nch % 2)
    mesh = plsc.VectorSubcoreMesh(core_axis_name="c", subcore_axis_name="s")

    @functools.partial(
        pl.kernel, mesh=mesh,
        out_type=jax.ShapeDtypeStruct((nout, NP, FC), jnp.float32),
        compiler_params=pltpu.CompilerParams(needs_layout_passes=False),
        scratch_types=[
            pltpu.VMEM((bedg,), jnp.int32),
            pltpu.VMEM((bedg,), jnp.float32),
            pltpu.VMEM((EB,), jnp.int32),
            pltpu.VMEM((EB,), jnp.int32),
            pltpu.VMEM((EB,), jnp.int32),
            pltpu.VMEM((EB,), jnp.int32),
            pltpu.VMEM((EB, FC), jnp.float32),
            pltpu.VMEM((EB, FC), jnp.float32),
            pltpu.VMEM((EB, FC), jnp.float32),
            pltpu.VMEM((EB, FC), jnp.float32),
            pltpu.VMEM_SHARED((NACC, FC), jnp.float32),
            pltpu.SemaphoreType.DMA,
            pltpu.SemaphoreType.DMA,
            pltpu.SemaphoreType.DMA,
            pltpu.SemaphoreType.DMA,
        ],
    )
    def k(tab_hbm, pk_hbm, w_hbm, z_hbm, out_hbm,
          pk_blk, w_blk, idxb0, idxb1, colb0, colb1,
          buf0, buf1, sbuf0, sbuf1, acc, sem0, sem1, semS0, semS1):
        cc = lax.axis_index("c")
        ss = lax.axis_index("s")

        def make_rows(lb, idxb, chunk_g):
            for g in range(EB // 16):
                pv = pk_blk[pl.ds(lb * EB + g * 16, 16)]
                rv = lax.shift_right_logical(pv, 14)
                idxb[pl.ds(g * 16, 16)] = rv * nch + chunk_g

        def make_cols(lb, colb):
            for g in range(EB // 16):
                pv = pk_blk[pl.ds(lb * EB + g * 16, 16)]
                colb[pl.ds(g * 16, 16)] = lax.bitwise_and(pv, PK - 1)

        def scale(lb, buf, sbuf):
            for e in range(EB):
                wsp = plsc.load_gather(
                    w_blk, [jnp.full((16,), lb * EB + e, jnp.int32)])
                for kk in range(FC // 16):
                    v = buf[e, pl.ds(kk * 16, 16)]
                    sbuf[e, pl.ds(kk * 16, 16)] = v * wsp

        def run_chunk(chunk_g, out_row, ebase0, nbat_j):
            pltpu.sync_copy(z_hbm, acc.at[pl.ds(ss * RPT, RPT)])
            plsc.subcore_barrier()

            def block_body(blk, carry1):
                bstart = blk * BB
                bcount = jnp.minimum(BB, nbat_j - bstart)
                pltpu.sync_copy(pk_hbm.at[pl.ds(ebase0 + bstart * EB, bedg)],
                                pk_blk)
                pltpu.sync_copy(w_hbm.at[pl.ds(ebase0 + bstart * EB, bedg)],
                                w_blk)
                make_rows(0, idxb0, chunk_g)
                pltpu.async_copy(tab_hbm.at[idxb0], buf0, sem0)
                make_rows(1, idxb1, chunk_g)
                pltpu.async_copy(tab_hbm.at[idxb1], buf1, sem1)

                def body(i, carry):
                    a = 2 * i

                    @pl.when(i > 0)
                    def _():
                        pltpu.make_async_copy(sbuf0, acc.at[colb0],
                                              semS0).wait()
                    pltpu.make_async_copy(tab_hbm.at[idxb0], buf0, sem0).wait()
                    scale(a, buf0, sbuf0)
                    make_rows(lax.rem(a + 2, bcount), idxb0, chunk_g)
                    pltpu.async_copy(tab_hbm.at[idxb0], buf0, sem0)
                    make_cols(a, colb0)
                    pltpu.async_copy(sbuf0, acc.at[colb0], semS0, add=True)

                    @pl.when(i > 0)
                    def _():
                        pltpu.make_async_copy(sbuf1, acc.at[colb1],
                                              semS1).wait()
                    pltpu.make_async_copy(tab_hbm.at[idxb1], buf1, sem1).wait()
                    scale(a + 1, buf1, sbuf1)
                    make_rows(lax.rem(a + 3, bcount), idxb1, chunk_g)
                    pltpu.async_copy(tab_hbm.at[idxb1], buf1, sem1)
                    make_cols(a + 1, colb1)
                    pltpu.async_copy(sbuf1, acc.at[colb1], semS1, add=True)
                    return carry

                lax.fori_loop(0, bcount // 2, body, 0)
                pltpu.make_async_copy(sbuf0, acc.at[colb0], semS0).wait()
                pltpu.make_async_copy(sbuf1, acc.at[colb1], semS1).wait()
                pltpu.make_async_copy(tab_hbm.at[idxb0], buf0, sem0).wait()
                pltpu.make_async_copy(tab_hbm.at[idxb1], buf1, sem1).wait()
                return carry1

            lax.fori_loop(0, (nbat_j + BB - 1) // BB, block_body, 0)
            plsc.subcore_barrier()
            pltpu.sync_copy(acc.at[pl.ds(ss * RPT, RPT)],
                            out_hbm.at[out_row, pl.ds(ss * RPT, RPT)])

        if nch % 2 == 0:
            def chunk_body(j, carry0):
                chunk_g = cc * n0 + j
                run_chunk(chunk_g, chunk_g, ss * ept, nbat)
                return carry0

            lax.fori_loop(0, n0, chunk_body, 0)
        else:
            def chunk_body(j, carry0):
                split = j == n0 - 1
                chunk_g = jnp.where(split, nch - 1, cc * (n0 - 1) + j)
                out_row = jnp.where(split, nch - 1 + cc, chunk_g)
                ebase0 = ss * ept + jnp.where(split & (cc == 1),
                                              half0 * EB, 0)
                nbat_j = jnp.where(split,
                                   jnp.where(cc == 0, half0, nbat - half0),
                                   nbat)
                run_chunk(chunk_g, out_row, ebase0, nbat_j)
                return carry0

            lax.fori_loop(0, n0, chunk_body, 0)

    return k(table, packed, w, zeros2d)


def _tc_prep(pdeg, Xt):
    def kfn(pd_ref, xt_ref, dinv_ref, t1_ref):
        d = jnp.sum(pd_ref[...], axis=0) + 1.0
        dv = lax.rsqrt(d)
        dinv_ref[...] = dv
        t1_ref[...] = xt_ref[...] * dv[:, None, None]

    return pl.pallas_call(
        kfn,
        grid=(NP // BN,),
        in_specs=[pl.BlockSpec((32, BN), lambda i: (0, i)),
                  pl.BlockSpec((BN, NCH1, FC), lambda i: (i, 0, 0))],
        out_specs=[pl.BlockSpec((BN,), lambda i: (i,)),
                   pl.BlockSpec((BN, NCH1, FC), lambda i: (i, 0, 0))],
        out_shape=[jax.ShapeDtypeStruct((NP,), jnp.float32),
                   jax.ShapeDtypeStruct((NP, NCH1, FC), jnp.float32)],
    )(pdeg, Xt)


def _tc_mid(Y, t1s, dinv, W1p, b1r, W2):
    def kfn(y_ref, x_ref, dv_ref, w1_ref, b1_ref, w2_ref, o_ref):
        dv = dv_ref[...][:, None]
        w1 = w1_ref[...]
        b1v = b1_ref[...]
        w2 = w2_ref[...]
        tpc1 = FC // FPAD
        tpc2 = FC // EMB
        for t in range(T):
            c1, o1 = t // tpc1, (t % tpc1) * FPAD
            if c1 < NCH1 - 1:
                ysl = y_ref[c1, :, o1:o1 + FPAD]
            else:
                ysl = (y_ref[NCH1 - 1, :, o1:o1 + FPAD]
                       + y_ref[NCH1, :, o1:o1 + FPAD])
            ax = (ysl + x_ref[:, c1, o1:o1 + FPAD]) * dv
            h1 = jnp.maximum(
                jnp.dot(ax, w1, preferred_element_type=jnp.float32) + b1v, 0.0)
            p = jnp.dot(h1, w2, preferred_element_type=jnp.float32)
            o2 = (t % tpc2) * EMB
            o_ref[:, t // tpc2, o2:o2 + EMB] = p * dv

    return pl.pallas_call(
        kfn,
        grid=(NP // BN,),
        in_specs=[pl.BlockSpec((NCH1 + 1, BN, FC), lambda i: (0, i, 0)),
                  pl.BlockSpec((BN, NCH1, FC), lambda i: (i, 0, 0)),
                  pl.BlockSpec((BN,), lambda i: (i,)),
                  pl.BlockSpec((FPAD, EMB), lambda i: (0, 0)),
                  pl.BlockSpec((1, EMB), lambda i: (0, 0)),
                  pl.BlockSpec((EMB, EMB), lambda i: (0, 0))],
        out_specs=pl.BlockSpec((BN, NCH2, FC), lambda i: (i, 0, 0)),
        out_shape=jax.ShapeDtypeStruct((NP, NCH2, FC), jnp.float32),
    )(Y, t1s, dinv, W1p, b1r, W2)


def _tc_final(Z, t2s, dinv, cbp, b2r, WihT, WhhT, bihr, bhhr,
              Wm1T, bm1r, Wm2T, bm2r):
    def kfn(z_ref, p_ref, dv_ref, cb_ref, b2_ref, wih_ref, whh_ref,
            bi_ref, bh_ref, wm1_ref, bm1_ref, wm2_ref, bm2_ref, o_ref):
        dv = dv_ref[...][:, None]
        add_t = b2_ref[...] + cb_ref[...]
        bias = bi_ref[...] + bh_ref[...]
        wih = wih_ref[...]
        whh = whh_ref[...]
        h = jnp.zeros((BN, H), jnp.float32)
        c = jnp.zeros((BN, H), jnp.float32)
        tpc2 = FC // EMB
        for t in range(T):
            sl = (t % tpc2) * EMB
            e = (z_ref[t // tpc2, :, sl:sl + EMB]
                 + p_ref[:, t // tpc2, sl:sl + EMB]) * dv + add_t
            g = (jnp.dot(e, wih, preferred_element_type=jnp.float32)
                 + jnp.dot(h, whh, preferred_element_type=jnp.float32) + bias)
            i_g = jax.nn.sigmoid(g[:, 0:H])
            f_g = jax.nn.sigmoid(g[:, H:2 * H])
            g_g = jnp.tanh(g[:, 2 * H:3 * H])
            o_g = jax.nn.sigmoid(g[:, 3 * H:4 * H])
            c = f_g * c + i_g * g_g
            h = o_g * jnp.tanh(c)
        z2 = jnp.maximum(
            jnp.dot(h, wm1_ref[...], preferred_element_type=jnp.float32)
            + bm1_ref[...], 0.0)
        pred = (jnp.dot(z2, wm2_ref[...], preferred_element_type=jnp.float32)
                + bm2_ref[...])
        o_ref[...] = pred[:, 0]

    return pl.pallas_call(
        kfn,
        grid=(NP // BN,),
        in_specs=[pl.BlockSpec((NCH2, BN, FC), lambda i: (0, i, 0)),
                  pl.BlockSpec((BN, NCH2, FC), lambda i: (i, 0, 0)),
                  pl.BlockSpec((BN,), lambda i: (i,)),
                  pl.BlockSpec((BN, EMB), lambda i: (i, 0)),
                  pl.BlockSpec((1, EMB), lambda i: (0, 0)),
                  pl.BlockSpec((EMB, 4 * H), lambda i: (0, 0)),
                  pl.BlockSpec((H, 4 * H), lambda i: (0, 0)),
                  pl.BlockSpec((1, 4 * H), lambda i: (0, 0)),
                  pl.BlockSpec((1, 4 * H), lambda i: (0, 0)),
                  pl.BlockSpec((H, H // 2), lambda i: (0, 0)),
                  pl.BlockSpec((1, H // 2), lambda i: (0, 0)),
                  pl.BlockSpec((H // 2, 1), lambda i: (0, 0)),
                  pl.BlockSpec((1, 1), lambda i: (0, 0))],
        out_specs=pl.BlockSpec((BN,), lambda i: (i,)),
        out_shape=jax.ShapeDtypeStruct((NP,), jnp.float32),
    )(Z, t2s, dinv, cbp, b2r, WihT, WhhT, bihr, bhhr, Wm1T, bm1r, Wm2T, bm2r)


def kernel(x, edge_index, edge_weight, W1, b1, W2, b2, county_bias,
           W_ih, W_hh, b_ih, b_hh, Wm1, bm1, Wm2, bm2):
    rows = edge_index[0].astype(jnp.int32)
    cols = edge_index[1].astype(jnp.int32)
    w = edge_weight.astype(jnp.float32)

    xp = jnp.pad(x, ((0, 0), (0, NP - N), (0, FPAD - FEAT)))
    Xt = xp.transpose(1, 0, 2).reshape(NP, NCH1, FC)
    z1 = jnp.zeros((NP,), jnp.float32)
    z2 = jnp.zeros((RPT, FC), jnp.float32)

    pdeg, packed = _sc_deg(rows, cols, w, z1)
    packed_p = jnp.pad(packed, (0, BB * EB))
    w_p = jnp.pad(w, (0, BB * EB))
    dinv, t1s = _tc_prep(pdeg, Xt)

    tab1 = t1s.reshape(NP * NCH1, FC)
    Y = _sc_spmm(tab1, packed_p, w_p, z2, NCH1)

    W1p = jnp.pad(W1, ((0, FPAD - FEAT), (0, 0)))
    t2s = _tc_mid(Y, t1s, dinv, W1p, b1[None], W2)

    tab2 = t2s.reshape(NP * NCH2, FC)
    Z = _sc_spmm(tab2, packed_p, w_p, z2, NCH2)

    cbp = jnp.pad(county_bias, ((0, NP - N), (0, 0)))
    preds = _tc_final(Z, t2s, dinv, cbp, b2[None], W_ih.T, W_hh.T,
                      b_ih[None], b_hh[None], Wm1.T, bm1[None], Wm2.T,
                      bm2[None])
    return preds[:N]
